# Initial kernel scaffold; baseline (speedup 1.0000x reference)
import functools

import jax
import jax.numpy as jnp
from jax import lax
from jax.experimental import pallas as pl
from jax.experimental.pallas import tpu as pltpu
from jax.experimental.pallas import tpu_sc as plsc

N_FIELDS = 26
VOCAB = 100000
EMB = 32
BATCH = 4096
DENSE = 256
EMB_FLAT = N_FIELDS * EMB

TOTAL_ROWS = N_FIELDS * BATCH
NUM_WORKERS = 32
ROWS_PER_WORKER = TOTAL_ROWS // NUM_WORKERS
CHUNK = 128
NCHUNKS = ROWS_PER_WORKER // CHUNK

_sc_mesh = plsc.VectorSubcoreMesh(core_axis_name="c", subcore_axis_name="s")


@functools.partial(
    pl.kernel,
    out_type=jax.ShapeDtypeStruct((TOTAL_ROWS, EMB), jnp.float32),
    mesh=_sc_mesh,
    scratch_types=[
        pltpu.VMEM((NCHUNKS, CHUNK), jnp.int32),
        pltpu.VMEM((CHUNK, EMB), jnp.float32),
        pltpu.VMEM((CHUNK, EMB), jnp.float32),
        pltpu.SemaphoreType.DMA,
        pltpu.SemaphoreType.DMA,
    ],
)
def _sc_gather(table_hbm, idx_hbm, out_hbm, idx_v, buf0, buf1, sem0, sem1):
    wid = lax.axis_index("s") * 2 + lax.axis_index("c")
    base = wid * ROWS_PER_WORKER
    pltpu.sync_copy(idx_hbm.at[wid], idx_v)

    bufs = (buf0, buf1)
    sems = (sem0, sem1)

    def start(j, b):
        return pltpu.async_copy(table_hbm.at[idx_v.at[j]], bufs[b], sems[b])

    start(0, 0)

    def body(t, carry):
        for b in range(2):
            j = 2 * t + b

            @pl.when(j + 1 < NCHUNKS)
            def _():
                start(j + 1, 1 - b)

            pltpu.make_async_copy(
                table_hbm.at[idx_v.at[j]], bufs[b], sems[b]
            ).wait()
            pltpu.sync_copy(
                bufs[b], out_hbm.at[pl.ds(base + j * CHUNK, CHUNK)]
            )
        return carry

    lax.fori_loop(0, NCHUNKS // 2, body, 0)


def _mlp_body(emb_ref, xd_ref, w1a_ref, w1b_ref, b1_ref, w2_ref, b2_ref,
              w3_ref, b3_ref, out_ref):
    h = jnp.dot(emb_ref[...], w1a_ref[...], preferred_element_type=jnp.float32)
    h = h + jnp.dot(xd_ref[...], w1b_ref[...],
                    preferred_element_type=jnp.float32)
    h = jnp.maximum(h + b1_ref[...], 0.0)
    h = jnp.dot(h, w2_ref[...], preferred_element_type=jnp.float32)
    h = jnp.maximum(h + b2_ref[...], 0.0)
    out_ref[...] = (
        jnp.dot(h, w3_ref[...], preferred_element_type=jnp.float32)
        + b3_ref[...]
    )


_BM = 1024


def _mlp(emb, xd, w1a, w1b, b1, w2, b2, w3, b3):
    grid = (BATCH // _BM,)
    return pl.pallas_call(
        _mlp_body,
        grid=grid,
        in_specs=[
            pl.BlockSpec((_BM, EMB_FLAT), lambda i: (i, 0)),
            pl.BlockSpec((_BM, DENSE), lambda i: (i, 0)),
            pl.BlockSpec((EMB_FLAT, 128), lambda i: (0, 0)),
            pl.BlockSpec((DENSE, 128), lambda i: (0, 0)),
            pl.BlockSpec((128,), lambda i: (0,)),
            pl.BlockSpec((128, 64), lambda i: (0, 0)),
            pl.BlockSpec((64,), lambda i: (0,)),
            pl.BlockSpec((64, 1), lambda i: (0, 0)),
            pl.BlockSpec((1,), lambda i: (0,)),
        ],
        out_specs=pl.BlockSpec((_BM, 1), lambda i: (i, 0)),
        out_shape=jax.ShapeDtypeStruct((BATCH, 1), jnp.float32),
    )(emb, xd, w1a, w1b, b1, w2, b2, w3, b3)


def kernel(x_cat, x_dense, emb_table, W1, b1, W2, b2, W3, b3):
    offs = (jnp.arange(N_FIELDS, dtype=jnp.int32) * VOCAB)[:, None]
    idx = (x_cat.astype(jnp.int32) + offs).T.reshape(
        NUM_WORKERS, NCHUNKS, CHUNK
    )
    table = emb_table.reshape(N_FIELDS * VOCAB, EMB)
    emb_rows = _sc_gather(table, idx)
    emb2 = emb_rows.reshape(BATCH, EMB_FLAT)
    out = _mlp(emb2, x_dense, W1[:EMB_FLAT], W1[EMB_FLAT:], b1, W2, b2, W3,
               b3)
    return out.reshape(BATCH)


# speedup vs baseline: 2.2024x; 2.2024x over previous
import functools

import jax
import jax.numpy as jnp
from jax import lax
from jax.experimental import pallas as pl
from jax.experimental.pallas import tpu as pltpu
from jax.experimental.pallas import tpu_sc as plsc

N_FIELDS = 26
VOCAB = 100000
EMB = 32
BATCH = 4096
DENSE = 256
EMB_FLAT = N_FIELDS * EMB

TOTAL_ROWS = N_FIELDS * BATCH
NUM_WORKERS = 32
ROWS_PER_WORKER = TOTAL_ROWS // NUM_WORKERS
CHUNK = 128
NCHUNKS = ROWS_PER_WORKER // CHUNK

@functools.cache
def _get_sc_gather():
    mesh = plsc.VectorSubcoreMesh(core_axis_name="c", subcore_axis_name="s")

    @functools.partial(
        pl.kernel,
        out_type=jax.ShapeDtypeStruct((TOTAL_ROWS, EMB), jnp.float32),
        mesh=mesh,
        scratch_types=[
            pltpu.VMEM((NCHUNKS, CHUNK), jnp.int32),
            pltpu.VMEM((CHUNK, EMB), jnp.float32),
            pltpu.VMEM((CHUNK, EMB), jnp.float32),
            pltpu.SemaphoreType.DMA,
            pltpu.SemaphoreType.DMA,
        ],
        compiler_params=pltpu.CompilerParams(use_tc_tiling_on_sc=False),
    )
    def _sc_gather(table_hbm, idx_hbm, out_hbm, idx_v, buf0, buf1, sem0, sem1):
        wid = lax.axis_index("s") * 2 + lax.axis_index("c")
        base = wid * ROWS_PER_WORKER
        pltpu.sync_copy(idx_hbm.at[wid], idx_v)

        bufs = (buf0, buf1)
        sems = (sem0, sem1)

        def start(j, b):
            return pltpu.async_copy(
                table_hbm.at[idx_v.at[j]], bufs[b], sems[b]
            )

        start(0, 0)

        def body(t, carry):
            for b in range(2):
                j = 2 * t + b

                @pl.when(j + 1 < NCHUNKS)
                def _():
                    start(j + 1, 1 - b)

                pltpu.make_async_copy(
                    table_hbm.at[idx_v.at[j]], bufs[b], sems[b]
                ).wait()
                pltpu.sync_copy(
                    bufs[b], out_hbm.at[pl.ds(base + j * CHUNK, CHUNK)]
                )
            return carry

        lax.fori_loop(0, NCHUNKS // 2, body, 0)

    return _sc_gather


def _mlp_body(emb_ref, xd_ref, w1a_ref, w1b_ref, b1_ref, w2_ref, b2_ref,
              w3_ref, b3_ref, out_ref):
    h = jnp.dot(emb_ref[...], w1a_ref[...], preferred_element_type=jnp.float32)
    h = h + jnp.dot(xd_ref[...], w1b_ref[...],
                    preferred_element_type=jnp.float32)
    h = jnp.maximum(h + b1_ref[...], 0.0)
    h = jnp.dot(h, w2_ref[...], preferred_element_type=jnp.float32)
    h = jnp.maximum(h + b2_ref[...], 0.0)
    out_ref[...] = (
        jnp.dot(h, w3_ref[...], preferred_element_type=jnp.float32)
        + b3_ref[...]
    )


_BM = 1024


def _mlp(emb, xd, w1a, w1b, b1, w2, b2, w3, b3):
    grid = (BATCH // _BM,)
    return pl.pallas_call(
        _mlp_body,
        grid=grid,
        in_specs=[
            pl.BlockSpec((_BM, EMB_FLAT), lambda i: (i, 0)),
            pl.BlockSpec((_BM, DENSE), lambda i: (i, 0)),
            pl.BlockSpec((EMB_FLAT, 128), lambda i: (0, 0)),
            pl.BlockSpec((DENSE, 128), lambda i: (0, 0)),
            pl.BlockSpec((128,), lambda i: (0,)),
            pl.BlockSpec((128, 64), lambda i: (0, 0)),
            pl.BlockSpec((64,), lambda i: (0,)),
            pl.BlockSpec((64, 1), lambda i: (0, 0)),
            pl.BlockSpec((1,), lambda i: (0,)),
        ],
        out_specs=pl.BlockSpec((_BM, 1), lambda i: (i, 0)),
        out_shape=jax.ShapeDtypeStruct((BATCH, 1), jnp.float32),
    )(emb, xd, w1a, w1b, b1, w2, b2, w3, b3)


def kernel(x_cat, x_dense, emb_table, W1, b1, W2, b2, W3, b3):
    offs = (jnp.arange(N_FIELDS, dtype=jnp.int32) * VOCAB)[:, None]
    idx = (x_cat.astype(jnp.int32) + offs).T.reshape(
        NUM_WORKERS, NCHUNKS, CHUNK
    )
    table = emb_table.reshape(N_FIELDS * VOCAB, EMB)
    emb_rows = _get_sc_gather()(table, idx)
    emb2 = emb_rows.reshape(BATCH, EMB_FLAT)
    out = _mlp(emb2, x_dense, W1[:EMB_FLAT], W1[EMB_FLAT:], b1, W2, b2, W3,
               b3)
    return out.reshape(BATCH)
